# baseline (speedup 1.0000x reference)
import jax
import jax.numpy as jnp
from jax import lax
from jax.experimental import pallas as pl
from jax.experimental.pallas import tpu as pltpu
from jax.experimental.pallas import tpu_sc as plsc

B, N, D = 1024, 77, 768
NW = 32
BPW = B // NW
CHUNKS = (0, 16, 32, 48, 61)


def _sc_body(tok_ref, emb_ref, pt_ref, pe_ref, out_ref,
             buf, tokv, pev, ptv, insem, outsem, fixsem):
    wid = lax.axis_index("s") * 2 + lax.axis_index("c")
    base = wid * BPW

    pltpu.make_async_copy(tok_ref.at[pl.ds(base, BPW)], tokv, insem).start()
    pltpu.make_async_copy(pe_ref, pev, outsem).start()
    pltpu.make_async_copy(pt_ref, ptv, fixsem).start()
    pltpu.make_async_copy(tok_ref.at[pl.ds(base, BPW)], tokv, insem).wait()
    pltpu.make_async_copy(pe_ref, pev, outsem).wait()
    pltpu.make_async_copy(pt_ref, ptv, fixsem).wait()

    def start_in(i):
        pltpu.make_async_copy(
            emb_ref.at[base + i], buf.at[i % 2], insem).start()

    def wait_in(i):
        pltpu.make_async_copy(
            emb_ref.at[base + i], buf.at[i % 2], insem).wait()

    def start_out(i):
        pltpu.make_async_copy(
            buf.at[i % 2], out_ref.at[base + i], outsem).start()

    def wait_out(i):
        pltpu.make_async_copy(
            buf.at[i % 2], out_ref.at[base + i], outsem).wait()

    start_in(0)
    for i in range(BPW):
        wait_in(i)
        start_out(i)
        if i + 1 < BPW:
            if i >= 1:
                wait_out(i - 1)
            start_in(i + 1)
    wait_out(BPW - 2)
    wait_out(BPW - 1)

    pt = ptv[...]
    lanes = lax.iota(jnp.int32, 16)

    def fix_row(i, carry):
        for start in CHUNKS:
            m0 = jnp.where(tokv[i, pl.ds(start, 16)] == pt, 1, 0)

            def cond(m):
                return jnp.max(m) > 0

            def body(m):
                inv = jnp.max(jnp.where(m > 0, 16 - lanes, 0))
                n = start + 16 - inv
                cp = pltpu.make_async_copy(
                    pev, out_ref.at[base + i, n], fixsem)
                cp.start()
                cp.wait()
                return jnp.where(lanes == (16 - inv), 0, m)

            lax.while_loop(cond, body, m0)
        return carry

    lax.fori_loop(0, BPW, fix_row, 0)


def sc_kernel(tokenized_text, embedded_text, placeholder_token, placeholder_embedding):
    pt_arr = jnp.full((16,), placeholder_token, jnp.int32)
    mesh = plsc.VectorSubcoreMesh(
        core_axis_name="c", subcore_axis_name="s",
        num_cores=2, num_subcores=16)
    k = pl.kernel(
        _sc_body,
        out_type=jax.ShapeDtypeStruct((B, N, D), jnp.float32),
        mesh=mesh,
        compiler_params=pltpu.CompilerParams(needs_layout_passes=False),
        scratch_types=[
            pltpu.VMEM((2, N, D), jnp.float32),
            pltpu.VMEM((BPW, N), jnp.int32),
            pltpu.VMEM((D,), jnp.float32),
            pltpu.VMEM((16,), jnp.int32),
            pltpu.SemaphoreType.DMA,
            pltpu.SemaphoreType.DMA,
            pltpu.SemaphoreType.DMA,
        ],
    )
    return k(tokenized_text, embedded_text, pt_arr, placeholder_embedding)


def kernel(tokenized_text, embedded_text, placeholder_token, placeholder_embedding):
    return sc_kernel(tokenized_text, embedded_text, placeholder_token,
                     placeholder_embedding)
